# Initial kernel scaffold; baseline (speedup 1.0000x reference)
import jax
import jax.numpy as jnp
from jax.experimental import pallas as pl


def kernel(x, s1, t1):
    raise NotImplementedError("write your pallas kernel here")


# speedup vs baseline: 13.1102x; 13.1102x over previous
import functools

import jax
import jax.numpy as jnp
from jax import lax
from jax.experimental import pallas as pl
from jax.experimental.pallas import tpu as pltpu
from jax.experimental.pallas import tpu_sc as plsc

B, N, D, K = 2, 10000, 128, 16
NC, NS = 2, 16
NW = NC * NS
C = 8
NCHUNK = (B * N) // C
LANES = 16
DB = D // LANES

_mesh = plsc.VectorSubcoreMesh(core_axis_name="c", subcore_axis_name="s")

_BCAST_DNUMS = lax.GatherDimensionNumbers(
    offset_dims=(), collapsed_slice_dims=(0,), start_index_map=(0,))


def _bcast_lane(v, k):
    idx = jnp.full((LANES, 1), k, jnp.int32)
    return lax.gather(v, idx, _BCAST_DNUMS, (1,),
                      mode=lax.GatherScatterMode.PROMISE_IN_BOUNDS)


@functools.partial(
    pl.kernel,
    out_type=jax.ShapeDtypeStruct((B * N, D), jnp.float32),
    mesh=_mesh,
    scratch_types=[
        pltpu.VMEM((C * K,), jnp.int32),
        pltpu.VMEM((C * K,), jnp.float32),
        pltpu.VMEM((C * K, D), jnp.float32),
        pltpu.VMEM((C, D), jnp.float32),
        pltpu.SemaphoreType.DMA,
    ],
)
def _hop(table_hbm, s_hbm, gidx_hbm, out_hbm, idx_v, s_v, rows_v, out_v, sem):
    cid = lax.axis_index("c")
    sid = lax.axis_index("s")
    wid = cid * NS + sid
    nchunks = NCHUNK // NW + jnp.where(wid < NCHUNK % NW, 1, 0)

    def chunk_body(j, carry):
        rbase = (j * NW + wid) * C
        ebase = rbase * K
        pltpu.sync_copy(gidx_hbm.at[pl.ds(ebase, C * K)], idx_v)
        pltpu.sync_copy(s_hbm.at[pl.ds(ebase, C * K)], s_v)
        pltpu.async_copy(table_hbm.at[idx_v], rows_v, sem).wait()
        for r in range(C):
            srow = s_v[pl.ds(r * K, K)]
            accs = [None] * DB
            for k in range(K):
                w = _bcast_lane(srow, k)
                for db in range(DB):
                    xv = rows_v[r * K + k, pl.ds(db * LANES, LANES)]
                    if accs[db] is None:
                        accs[db] = w * xv
                    else:
                        accs[db] = accs[db] + w * xv
            for db in range(DB):
                out_v[r, pl.ds(db * LANES, LANES)] = accs[db]
        pltpu.sync_copy(out_v, out_hbm.at[pl.ds(rbase, C)])
        return carry

    lax.fori_loop(0, nchunks, chunk_body, 0)


def kernel(x, s1, t1):
    xf = x.reshape(B * N, D)
    offs = (jnp.arange(B, dtype=jnp.int32) * N)[:, None, None]
    gidx = (t1.astype(jnp.int32) + offs).reshape(B * N * K)
    sf = s1.reshape(B * N * K)
    x1 = _hop(xf, sf, gidx)
    x2 = _hop(x1, sf, gidx)
    return jnp.stack([xf, x1, x2], axis=0).reshape(3, B, N, D).transpose(1, 0, 2, 3)
